# baseline (speedup 1.0000x reference)
import jax
import jax.numpy as jnp
from jax.experimental import pallas as pl
from jax.experimental.pallas import tpu as pltpu

_B, _L, _C, _CURVE = 1024, 50, 3, 128
_FIN = _C * _CURVE
_DM = 128
_DLLM = 768
_E = 16
_TOPP = 2.0 / 16.0
_EPS = 1e-9
_BB = 64


def _moe_body(dkp_ref, gate_w_ref, gate_b_ref, flat_ref, ew_ref, eb_ref,
              gen_w_ref, gen_b_ref, out_ref):
    dkp = dkp_ref[...]
    logits = jnp.dot(dkp, gate_w_ref[...],
                     preferred_element_type=jnp.float32) + gate_b_ref[...]
    probs = jax.nn.softmax(logits, axis=-1)

    col = jax.lax.broadcasted_iota(jnp.int32, probs.shape, 1)
    cum_excl = jnp.zeros_like(probs)
    for j in range(_E):
        pj = probs[:, j:j + 1]
        before = (pj > probs) | ((pj == probs) & (j < col))
        cum_excl = cum_excl + jnp.where(before, pj, 0.0)
    mask = ((cum_excl + probs) < _TOPP) | (cum_excl <= 0.0)
    g = jnp.where(mask, probs, 0.0)
    g = g / (jnp.sum(g, axis=1, keepdims=True) + _EPS)

    ew = ew_ref[...]
    mixed = jax.lax.dot_general(g, ew, (((1,), (0,)), ((), ())),
                                preferred_element_type=jnp.float32)
    flat = flat_ref[...]
    comb = jax.lax.dot_general(flat, mixed, (((2,), (1,)), ((0,), (0,))),
                               preferred_element_type=jnp.float32)
    comb = comb + jnp.dot(g, eb_ref[...],
                          preferred_element_type=jnp.float32)[:, None, :]
    comb = comb.astype(jnp.bfloat16)
    gen = jax.lax.dot_general(flat, gen_w_ref[...], (((2,), (0,)), ((), ())),
                              preferred_element_type=jnp.float32)
    out_ref[...] = (gen + gen_b_ref[...][None, :, :]) + comb.astype(jnp.float32)


def kernel(cycle_curve_data, DKP_embeddings, gate_W, gate_b, expert_W,
           expert_b, gen_W, gen_b):
    flat = cycle_curve_data.reshape(_B, _L, _FIN)
    gate_b2 = gate_b.reshape(1, _E)
    gen_w2 = gen_W.reshape(_FIN, _DM)
    gen_b2 = gen_b.reshape(1, _DM)

    grid = (_B // _BB,)
    out = pl.pallas_call(
        _moe_body,
        grid=grid,
        in_specs=[
            pl.BlockSpec((_BB, _DLLM), lambda i: (i, 0)),
            pl.BlockSpec((_DLLM, _E), lambda i: (0, 0)),
            pl.BlockSpec((1, _E), lambda i: (0, 0)),
            pl.BlockSpec((_BB, _L, _FIN), lambda i: (i, 0, 0)),
            pl.BlockSpec((_E, _FIN, _DM), lambda i: (0, 0, 0)),
            pl.BlockSpec((_E, _DM), lambda i: (0, 0)),
            pl.BlockSpec((_FIN, _DM), lambda i: (0, 0)),
            pl.BlockSpec((1, _DM), lambda i: (0, 0)),
        ],
        out_specs=pl.BlockSpec((_BB, _L, _DM), lambda i: (i, 0, 0)),
        out_shape=jax.ShapeDtypeStruct((_B, _L, _DM), jnp.float32),
    )(DKP_embeddings, gate_W, gate_b2, flat, expert_W, expert_b, gen_w2,
      gen_b2)
    return out
